# baseline (speedup 1.0000x reference)
import jax
import jax.numpy as jnp
from jax import lax
from jax.experimental import pallas as pl
from jax.experimental.pallas import tpu as pltpu
from jax.experimental.pallas import tpu_sc as plsc

VOCAB = 100000
DIM = 300
DIMP = 304
N_SPECIAL = 5

NC = 2
NS = 16
NW = NC * NS
LANES = 16

TOKENS = 1024 * 200
N_PER = TOKENS // NW
CHUNK = 128
NCHUNK = N_PER // CHUNK


def _row_copy(dst_ref, dst_row, src_ref, src_row):
    for c in range(0, DIMP, LANES):
        dst_ref[dst_row, pl.ds(c, LANES)] = src_ref[src_row, pl.ds(c, LANES)]


def _fixup_chunk(idx_v, g, aux_v, buf):
    lanes = lax.broadcasted_iota(jnp.int32, (LANES,), 0)

    def group(k, _):
        ids16 = idx_v[g, pl.ds(k * LANES, LANES)]
        is_pad = ids16 == 0
        is_spec = ids16 >= (VOCAB - N_SPECIAL)
        m = jnp.logical_or(is_pad, is_spec)
        mi = jnp.where(m, 1, 0)
        cnt = jnp.sum(mi)

        @pl.when(cnt > 0)
        def _():
            a16 = jnp.where(is_spec, ids16 - (VOCAB - N_SPECIAL) + 1, 0)

            def lane(i, _):
                sel = jnp.where(lanes == i, 1, 0)
                flag = jnp.sum(sel * mi)

                @pl.when(flag > 0)
                def _():
                    a = jnp.sum(sel * a16)
                    t = k * LANES + i
                    _row_copy(buf, t, aux_v, a)

                return 0

            lax.fori_loop(0, LANES, lane, 0)

        return 0

    lax.fori_loop(0, CHUNK // LANES, group, 0)


def _sc_kernel(idx_hbm, aux_hbm, table_hbm, out_hbm,
               idx_v, aux_v, rows_v, gsem, ssem):
    cid = lax.axis_index("c")
    sid = lax.axis_index("s")
    wid = sid * NC + cid
    cbase = wid * NCHUNK

    pltpu.sync_copy(idx_hbm.at[pl.ds(cbase, NCHUNK)], idx_v)
    pltpu.sync_copy(aux_hbm, aux_v)

    @pl.loop(0, NCHUNK)
    def _(g):
        pltpu.async_copy(table_hbm.at[idx_v.at[g]], rows_v, gsem).wait()
        _fixup_chunk(idx_v, g, aux_v, rows_v)
        pltpu.async_copy(
            rows_v, out_hbm.at[pl.ds((cbase + g) * CHUNK, CHUNK)], ssem).wait()


@jax.jit
def _run(idx2d, aux, table_p):
    mesh = plsc.VectorSubcoreMesh(
        core_axis_name="c", subcore_axis_name="s",
        num_cores=NC, num_subcores=NS)
    fn = pl.kernel(
        _sc_kernel,
        out_type=jax.ShapeDtypeStruct((TOKENS, DIMP), jnp.float32),
        mesh=mesh,
        scratch_types=[
            pltpu.VMEM((NCHUNK, CHUNK), jnp.int32),
            pltpu.VMEM((8, DIMP), jnp.float32),
            pltpu.VMEM((CHUNK, DIMP), jnp.float32),
            pltpu.SemaphoreType.DMA,
            pltpu.SemaphoreType.DMA,
        ],
        compiler_params=pltpu.CompilerParams(
            use_tc_tiling_on_sc=False, needs_layout_passes=False),
    )
    return fn(idx2d, aux, table_p)


def kernel(token_ids, table, w_special):
    B, L = token_ids.shape
    idx2d = token_ids.reshape(TOKENS // CHUNK, CHUNK).astype(jnp.int32)
    s = lax.optimization_barrier(jnp.float32(1.0))
    table_p = jnp.pad(table * s, ((0, 0), (0, DIMP - DIM)))
    aux = jnp.zeros((8, DIMP), jnp.float32).at[1:1 + N_SPECIAL, :DIM].set(w_special)
    out = _run(idx2d, aux, table_p)
    return (out[:, :DIM] * s).reshape(B, L, DIM)


# speedup vs baseline: 1.2206x; 1.2206x over previous
import jax
import jax.numpy as jnp
from jax import lax
from jax.experimental import pallas as pl
from jax.experimental.pallas import tpu as pltpu
from jax.experimental.pallas import tpu_sc as plsc

VOCAB = 100000
DIM = 300
DIMP = 304
N_SPECIAL = 5

NC = 2
NS = 16
NW = NC * NS
LANES = 16

TOKENS = 1024 * 200
N_PER = TOKENS // NW
CHUNK = 128
NCHUNK = N_PER // CHUNK


def _row_copy(dst_ref, dst_row, src_ref, src_row):
    for c in range(0, DIMP, LANES):
        dst_ref[dst_row, pl.ds(c, LANES)] = src_ref[src_row, pl.ds(c, LANES)]


def _fixup_chunk(idx_v, g, aux_v, buf):
    lanes = lax.broadcasted_iota(jnp.int32, (LANES,), 0)

    def group(k, _):
        ids16 = idx_v[g, pl.ds(k * LANES, LANES)]
        is_pad = ids16 == 0
        is_spec = ids16 >= (VOCAB - N_SPECIAL)
        m = jnp.logical_or(is_pad, is_spec)
        mi = jnp.where(m, 1, 0)
        cnt = jnp.sum(mi)

        @pl.when(cnt > 0)
        def _():
            a16 = jnp.where(is_spec, ids16 - (VOCAB - N_SPECIAL) + 1, 0)

            def lane(i, _):
                sel = jnp.where(lanes == i, 1, 0)
                flag = jnp.sum(sel * mi)

                @pl.when(flag > 0)
                def _():
                    a = jnp.sum(sel * a16)
                    t = k * LANES + i
                    _row_copy(buf, t, aux_v, a)

                return 0

            lax.fori_loop(0, LANES, lane, 0)

        return 0

    lax.fori_loop(0, CHUNK // LANES, group, 0)


def _sc_kernel(idx_hbm, aux_hbm, table_hbm, out_hbm,
               idx_v, aux_v, rows0, rows1, gsem0, gsem1, ssem0, ssem1):
    cid = lax.axis_index("c")
    sid = lax.axis_index("s")
    wid = sid * NC + cid
    cbase = wid * NCHUNK

    pltpu.sync_copy(idx_hbm.at[pl.ds(cbase, NCHUNK)], idx_v)
    pltpu.sync_copy(aux_hbm, aux_v)

    bufs = (rows0, rows1)
    gsems = (gsem0, gsem1)
    ssems = (ssem0, ssem1)

    def gather_desc(g, b):
        return pltpu.make_async_copy(table_hbm.at[idx_v.at[g]], bufs[b], gsems[b])

    def store_desc(g, b):
        return pltpu.make_async_copy(
            bufs[b], out_hbm.at[pl.ds((cbase + g) * CHUNK, CHUNK)], ssems[b])

    gather_desc(0, 0).start()
    gather_desc(1, 1).start()

    @pl.loop(0, NCHUNK, step=2)
    def _(g0):
        for j in range(2):
            g = g0 + j
            gather_desc(g, j).wait()
            _fixup_chunk(idx_v, g, aux_v, bufs[j])
            store_desc(g, j).start()

            @pl.when(g + 2 < NCHUNK)
            def _():
                store_desc(g, j).wait()
                gather_desc(g + 2, j).start()

    store_desc(NCHUNK - 2, 0).wait()
    store_desc(NCHUNK - 1, 1).wait()


@jax.jit
def _run(idx2d, aux, table_p):
    mesh = plsc.VectorSubcoreMesh(
        core_axis_name="c", subcore_axis_name="s",
        num_cores=NC, num_subcores=NS)
    fn = pl.kernel(
        _sc_kernel,
        out_type=jax.ShapeDtypeStruct((TOKENS, DIMP), jnp.float32),
        mesh=mesh,
        scratch_types=[
            pltpu.VMEM((NCHUNK, CHUNK), jnp.int32),
            pltpu.VMEM((8, DIMP), jnp.float32),
            pltpu.VMEM((CHUNK, DIMP), jnp.float32),
            pltpu.VMEM((CHUNK, DIMP), jnp.float32),
            pltpu.SemaphoreType.DMA,
            pltpu.SemaphoreType.DMA,
            pltpu.SemaphoreType.DMA,
            pltpu.SemaphoreType.DMA,
        ],
        compiler_params=pltpu.CompilerParams(
            use_tc_tiling_on_sc=False, needs_layout_passes=False),
    )
    return fn(idx2d, aux, table_p)


def kernel(token_ids, table, w_special):
    B, L = token_ids.shape
    idx2d = token_ids.reshape(TOKENS // CHUNK, CHUNK).astype(jnp.int32)
    table_p = jnp.pad(table, ((0, 0), (0, DIMP - DIM)))
    aux = jnp.zeros((8, DIMP), jnp.float32).at[1:1 + N_SPECIAL, :DIM].set(w_special)
    out = _run(idx2d, aux, table_p)
    return out[:, :DIM].reshape(B, L, DIM)
